# baseline (speedup 1.0000x reference)
import functools

import jax
import jax.numpy as jnp
from jax import lax
from jax.experimental import pallas as pl
from jax.experimental.pallas import tpu as pltpu
from jax.experimental.pallas import tpu_sc as plsc

N_NODES = 10000
N_EDGES = 320000
D_IN = 128
N_SUBCORES = 16
CHUNK = 128
PHASE_CHUNKS = 79
N_PHASES = 2
EDGES_PER_TILE = N_PHASES * PHASE_CHUNKS * CHUNK
E_PAD = N_SUBCORES * EDGES_PER_TILE
CBUF = (PHASE_CHUNKS + 2) * CHUNK
HALF = N_NODES // 2
HALF_PAD = 5120
ROWS_PER_TILE = HALF_PAD // N_SUBCORES
TRASH = HALF_PAD - 1
DEG_PAD = 10016
VECS = CHUNK // 16


@functools.partial(
    pl.kernel,
    out_type=(
        jax.ShapeDtypeStruct((2, HALF_PAD, D_IN), jnp.float32),
        jax.ShapeDtypeStruct((N_SUBCORES, DEG_PAD), jnp.float32),
    ),
    mesh=plsc.VectorSubcoreMesh(core_axis_name="c", subcore_axis_name="s"),
    compiler_params=pltpu.CompilerParams(needs_layout_passes=False),
    scratch_types=[
        pltpu.VMEM((PHASE_CHUNKS, CHUNK), jnp.int32),
        pltpu.VMEM((PHASE_CHUNKS, CHUNK), jnp.int32),
        pltpu.VMEM((CBUF,), jnp.int32),
        pltpu.VMEM((CBUF,), jnp.int32),
        pltpu.VMEM((CHUNK, D_IN), jnp.float32),
        pltpu.VMEM((CHUNK, D_IN), jnp.float32),
        pltpu.VMEM((DEG_PAD,), jnp.float32),
        pltpu.VMEM_SHARED((HALF_PAD, D_IN), jnp.float32),
        pltpu.SemaphoreType.DMA,
        pltpu.SemaphoreType.DMA,
    ],
)
def _sc_aggregate(h_hbm, src_r, dst_r, zeros2d, zeros1d, out_feat, out_deg,
                  src_v, dst_v, src_c, dst_c, buf_a, buf_b, deg_v, acc,
                  sem_ga, sem_gb):
    c = lax.axis_index("c")
    s = lax.axis_index("s")
    row0 = s * ROWS_PER_TILE

    pltpu.sync_copy(zeros2d.at[pl.ds(row0, ROWS_PER_TILE)],
                    acc.at[pl.ds(row0, ROWS_PER_TILE)])
    pltpu.sync_copy(zeros1d, deg_v)
    plsc.subcore_barrier()

    ones16 = jnp.ones((16,), jnp.float32)
    trash16 = jnp.full((16,), TRASH, jnp.int32)
    zero16 = jnp.zeros((16,), jnp.int32)
    base = c * HALF

    for p in range(N_PHASES):
        pltpu.sync_copy(src_r.at[s, p], src_v)
        pltpu.sync_copy(dst_r.at[s, p], dst_v)

        def pre_body(e, cnt):
            j = e // VECS
            k = e - j * VECS
            vd = dst_v[j, pl.ds(k * 16, 16)]
            vs = src_v[j, pl.ds(k * 16, 16)]

            @pl.when(c == 0)
            def _():
                plsc.addupdate_scatter(deg_v, [vd], ones16)

            local = vd - base
            valid = (local >= 0) & (local < HALF)
            plsc.store_compressed(dst_c.at[pl.ds(cnt, 16)], local, mask=valid)
            plsc.store_compressed(src_c.at[pl.ds(cnt, 16)], vs, mask=valid)
            return cnt + jnp.sum(jnp.where(valid, 1, 0))

        with jax.named_scope("compact"):
            cnt = lax.fori_loop(0, PHASE_CHUNKS * VECS, pre_body, 0)

        for k in range(2 * VECS):
            dst_c[pl.ds(cnt + k * 16, 16)] = trash16
            src_c[pl.ds(cnt + k * 16, 16)] = zero16

        n_pairs = jnp.maximum((cnt + 2 * CHUNK - 1) // (2 * CHUNK), 1)
        last_even = 2 * n_pairs - 2

        def gather(j, buf, sem):
            pltpu.async_copy(h_hbm.at[src_c.at[pl.ds(j * CHUNK, CHUNK)]],
                             buf, sem)

        def gather_wait(j, buf, sem):
            pltpu.make_async_copy(h_hbm.at[src_c.at[pl.ds(j * CHUNK, CHUNK)]],
                                  buf, sem).wait()

        def scatter_sync(j, buf):
            pltpu.sync_copy(buf, acc.at[dst_c.at[pl.ds(j * CHUNK, CHUNK)]],
                            add=True)

        def body(jj, carry):
            j0 = 2 * jj
            j1 = j0 + 1
            gather(j1, buf_b, sem_gb)
            gather_wait(j0, buf_a, sem_ga)
            scatter_sync(j0, buf_a)
            gather(jnp.minimum(j0 + 2, last_even), buf_a, sem_ga)
            gather_wait(j1, buf_b, sem_gb)
            scatter_sync(j1, buf_b)
            return carry

        with jax.named_scope("mainloop"):
            gather(0, buf_a, sem_ga)
            lax.fori_loop(0, n_pairs, body, 0)
            gather_wait(last_even, buf_a, sem_ga)

    @pl.when(c == 0)
    def _():
        pltpu.sync_copy(deg_v, out_deg.at[s])

    plsc.subcore_barrier()
    pltpu.sync_copy(acc.at[pl.ds(row0, ROWS_PER_TILE)],
                    out_feat.at[c, pl.ds(row0, ROWS_PER_TILE)])


def _tc_body(h_ref, p_ref, deg_ref, wt1_ref, wt2_ref, b_ref, o_ref):
    summed = p_ref[0]
    deg = jnp.sum(deg_ref[...], axis=1)
    hn = summed / jnp.maximum(deg, 1.0)[:, None]
    o_ref[...] = (
        jnp.dot(h_ref[...], wt1_ref[...], preferred_element_type=jnp.float32)
        + jnp.dot(hn, wt2_ref[...], preferred_element_type=jnp.float32)
        + b_ref[...]
    )


BLK = 1000


def kernel(h, edge_index, W, b):
    n, d = h.shape
    src = edge_index[0].astype(jnp.int32)
    dst = edge_index[1].astype(jnp.int32)
    pad = E_PAD - N_EDGES
    src = jnp.concatenate([src, jnp.zeros((pad,), jnp.int32)])
    dst = jnp.concatenate([dst, jnp.full((pad,), N_NODES, jnp.int32)])
    src = src.reshape(N_SUBCORES, N_PHASES, PHASE_CHUNKS, CHUNK)
    dst = dst.reshape(N_SUBCORES, N_PHASES, PHASE_CHUNKS, CHUNK)
    zeros2d = jnp.zeros((HALF_PAD, d), jnp.float32)
    zeros1d = jnp.zeros((DEG_PAD,), jnp.float32)

    partials, deg_parts = _sc_aggregate(h, src, dst, zeros2d, zeros1d)

    wt = W.T
    wt1 = wt[:d]
    wt2 = wt[d:]
    b2 = b.reshape(1, -1)

    nblk_half = HALF // BLK
    out = pl.pallas_call(
        _tc_body,
        grid=(n // BLK,),
        in_specs=[
            pl.BlockSpec((BLK, d), lambda i: (i, 0)),
            pl.BlockSpec((1, BLK, d), lambda i: (i // nblk_half, i % nblk_half, 0)),
            pl.BlockSpec((BLK, N_SUBCORES), lambda i: (i, 0)),
            pl.BlockSpec((d, d), lambda i: (0, 0)),
            pl.BlockSpec((d, d), lambda i: (0, 0)),
            pl.BlockSpec((1, d), lambda i: (0, 0)),
        ],
        out_specs=pl.BlockSpec((BLK, d), lambda i: (i, 0)),
        out_shape=jax.ShapeDtypeStruct((n, d), jnp.float32),
    )(h, partials, deg_parts[:, :n].T, wt1, wt2, b2)
    return out


# speedup vs baseline: 1.1735x; 1.1735x over previous
import functools

import jax
import jax.numpy as jnp
from jax import lax
from jax.experimental import pallas as pl
from jax.experimental.pallas import tpu as pltpu
from jax.experimental.pallas import tpu_sc as plsc

N_NODES = 10000
N_EDGES = 320000
D_IN = 128
N_SUBCORES = 16
CHUNK = 128
EDGES_PER_TILE = N_EDGES // N_SUBCORES
PH0 = 10112
PH1 = EDGES_PER_TILE - PH0
CBUF = 10240
HALF = N_NODES // 2
HALF_PAD = 5120
ROWS_PER_TILE = HALF_PAD // N_SUBCORES
TRASH = HALF_PAD - 1


@functools.partial(
    pl.kernel,
    out_type=(
        jax.ShapeDtypeStruct((2, HALF_PAD, D_IN), jnp.float32),
        jax.ShapeDtypeStruct((N_SUBCORES, N_NODES), jnp.float32),
    ),
    mesh=plsc.VectorSubcoreMesh(core_axis_name="c", subcore_axis_name="s"),
    compiler_params=pltpu.CompilerParams(needs_layout_passes=False),
    scratch_types=[
        pltpu.VMEM((PH0,), jnp.int32),
        pltpu.VMEM((PH0,), jnp.int32),
        pltpu.VMEM((CBUF,), jnp.int32),
        pltpu.VMEM((CBUF,), jnp.int32),
        pltpu.VMEM((CHUNK, D_IN), jnp.float32),
        pltpu.VMEM((N_NODES,), jnp.float32),
        pltpu.VMEM_SHARED((HALF_PAD, D_IN), jnp.float32),
        pltpu.SemaphoreType.DMA,
    ],
)
def _sc_aggregate(h_hbm, src_hbm, dst_hbm, zeros2d, zeros1d, out_feat, out_deg,
                  src_v, dst_v, src_c, dst_c, buf, deg_v, acc, sem):
    c = lax.axis_index("c")
    s = lax.axis_index("s")
    row0 = s * ROWS_PER_TILE
    e0 = s * EDGES_PER_TILE

    with jax.named_scope("prologue"):
        pltpu.sync_copy(zeros2d.at[pl.ds(row0, ROWS_PER_TILE)],
                        acc.at[pl.ds(row0, ROWS_PER_TILE)])
        pltpu.sync_copy(zeros1d, deg_v)
    plsc.subcore_barrier()

    ones16 = jnp.ones((16,), jnp.float32)
    trash16 = jnp.full((16,), TRASH, jnp.int32)
    zero16 = jnp.zeros((16,), jnp.int32)
    base = c * HALF

    for start, n_edges in ((0, PH0), (PH0, PH1)):
        with jax.named_scope("stage"):
            pltpu.sync_copy(src_hbm.at[pl.ds(e0 + start, n_edges)],
                            src_v.at[pl.ds(0, n_edges)])
            pltpu.sync_copy(dst_hbm.at[pl.ds(e0 + start, n_edges)],
                            dst_v.at[pl.ds(0, n_edges)])

        def pre_body(v, cnt):
            vd = dst_v[pl.ds(v * 16, 16)]
            vs = src_v[pl.ds(v * 16, 16)]

            @pl.when(c == 0)
            def _():
                plsc.addupdate_scatter(deg_v, [vd], ones16)

            local = vd - base
            valid = (local >= 0) & (local < HALF)
            plsc.store_compressed(dst_c.at[pl.ds(cnt, 16)], local, mask=valid)
            plsc.store_compressed(src_c.at[pl.ds(cnt, 16)], vs, mask=valid)
            return cnt + jnp.sum(jnp.where(valid, 1, 0))

        with jax.named_scope("compact"):
            cnt = lax.fori_loop(0, n_edges // 16, pre_body, 0)

        for k in range(CHUNK // 16):
            dst_c[pl.ds(cnt + k * 16, 16)] = trash16
            src_c[pl.ds(cnt + k * 16, 16)] = zero16

        n_ch = (cnt + CHUNK - 1) // CHUNK

        def body(j, carry):
            pltpu.async_copy(h_hbm.at[src_c.at[pl.ds(j * CHUNK, CHUNK)]],
                             buf, sem).wait()
            pltpu.sync_copy(buf, acc.at[dst_c.at[pl.ds(j * CHUNK, CHUNK)]],
                            add=True)
            return carry

        with jax.named_scope("mainloop"):
            lax.fori_loop(0, n_ch, body, 0)

    with jax.named_scope("degout"):
        @pl.when(c == 0)
        def _():
            pltpu.sync_copy(deg_v, out_deg.at[s])

    plsc.subcore_barrier()
    with jax.named_scope("writeout"):
        pltpu.sync_copy(acc.at[pl.ds(row0, ROWS_PER_TILE)],
                        out_feat.at[c, pl.ds(row0, ROWS_PER_TILE)])


def _tc_body(h_ref, p_ref, deg_ref, wt1_ref, wt2_ref, b_ref, o_ref):
    summed = p_ref[0]
    deg = jnp.sum(deg_ref[...], axis=1)
    hn = summed / jnp.maximum(deg, 1.0)[:, None]
    o_ref[...] = (
        jnp.dot(h_ref[...], wt1_ref[...], preferred_element_type=jnp.float32)
        + jnp.dot(hn, wt2_ref[...], preferred_element_type=jnp.float32)
        + b_ref[...]
    )


BLK = 1000


def kernel(h, edge_index, W, b):
    n, d = h.shape
    src = edge_index[0].astype(jnp.int32)
    dst = edge_index[1].astype(jnp.int32)
    zeros2d = jnp.zeros((HALF_PAD, d), jnp.float32)
    zeros1d = jnp.zeros((n,), jnp.float32)

    partials, deg_parts = _sc_aggregate(h, src, dst, zeros2d, zeros1d)

    wt = W.T
    wt1 = wt[:d]
    wt2 = wt[d:]
    b2 = b.reshape(1, -1)

    nblk_half = HALF // BLK
    out = pl.pallas_call(
        _tc_body,
        grid=(n // BLK,),
        in_specs=[
            pl.BlockSpec((BLK, d), lambda i: (i, 0)),
            pl.BlockSpec((1, BLK, d), lambda i: (i // nblk_half, i % nblk_half, 0)),
            pl.BlockSpec((BLK, N_SUBCORES), lambda i: (i, 0)),
            pl.BlockSpec((d, d), lambda i: (0, 0)),
            pl.BlockSpec((d, d), lambda i: (0, 0)),
            pl.BlockSpec((1, d), lambda i: (0, 0)),
        ],
        out_specs=pl.BlockSpec((BLK, d), lambda i: (i, 0)),
        out_shape=jax.ShapeDtypeStruct((n, d), jnp.float32),
    )(h, partials, deg_parts.T, wt1, wt2, b2)
    return out


# speedup vs baseline: 1.2171x; 1.0372x over previous
import functools

import jax
import jax.numpy as jnp
from jax import lax
from jax.experimental import pallas as pl
from jax.experimental.pallas import tpu as pltpu
from jax.experimental.pallas import tpu_sc as plsc

N_NODES = 10000
N_EDGES = 320000
D_IN = 128
N_SUBCORES = 16
CHUNK = 128
PHASE_CHUNKS = 79
N_PHASES = 2
EDGES_PER_TILE = N_PHASES * PHASE_CHUNKS * CHUNK
E_PAD = N_SUBCORES * EDGES_PER_TILE
CBUF = (PHASE_CHUNKS + 1) * CHUNK
HALF = N_NODES // 2
HALF_PAD = 5120
ROWS_PER_TILE = HALF_PAD // N_SUBCORES
TRASH = HALF_PAD - 1
DEG_PAD = 10016


@functools.partial(
    pl.kernel,
    out_type=(
        jax.ShapeDtypeStruct((2, HALF_PAD, D_IN), jnp.float32),
        jax.ShapeDtypeStruct((N_SUBCORES, DEG_PAD), jnp.float32),
    ),
    mesh=plsc.VectorSubcoreMesh(core_axis_name="c", subcore_axis_name="s"),
    compiler_params=pltpu.CompilerParams(needs_layout_passes=False),
    scratch_types=[
        pltpu.VMEM((PHASE_CHUNKS, CHUNK), jnp.int32),
        pltpu.VMEM((PHASE_CHUNKS, CHUNK), jnp.int32),
        pltpu.VMEM((CBUF,), jnp.int32),
        pltpu.VMEM((CBUF,), jnp.int32),
        pltpu.VMEM((CHUNK, D_IN), jnp.float32),
        pltpu.VMEM((DEG_PAD,), jnp.float32),
        pltpu.VMEM_SHARED((HALF_PAD, D_IN), jnp.float32),
        pltpu.SemaphoreType.DMA,
        pltpu.SemaphoreType.DMA,
        pltpu.SemaphoreType.DMA,
    ],
)
def _sc_aggregate(h_hbm, src_r, dst_r, zeros2d, zeros1d, out_feat, out_deg,
                  src_v, dst_v, src_c, dst_c, buf, deg_v, acc,
                  sem, sem_b, sem_c):
    c = lax.axis_index("c")
    s = lax.axis_index("s")
    row0 = s * ROWS_PER_TILE

    with jax.named_scope("prologue"):
        pltpu.async_copy(zeros2d.at[pl.ds(row0, ROWS_PER_TILE)],
                         acc.at[pl.ds(row0, ROWS_PER_TILE)], sem)
        pltpu.async_copy(src_r.at[s, 0], src_v, sem_b)
        pltpu.async_copy(dst_r.at[s, 0], dst_v, sem_c)
        pltpu.sync_copy(zeros1d, deg_v)
        pltpu.make_async_copy(zeros2d.at[pl.ds(row0, ROWS_PER_TILE)],
                              acc.at[pl.ds(row0, ROWS_PER_TILE)], sem).wait()
        pltpu.make_async_copy(src_r.at[s, 0], src_v, sem_b).wait()
        pltpu.make_async_copy(dst_r.at[s, 0], dst_v, sem_c).wait()
    plsc.subcore_barrier()

    ones16 = jnp.ones((16,), jnp.float32)
    trash16 = jnp.full((16,), TRASH, jnp.int32)
    zero16 = jnp.zeros((16,), jnp.int32)
    base = c * HALF

    for p in range(N_PHASES):
        if p > 0:
            with jax.named_scope("stage"):
                st1 = pltpu.async_copy(src_r.at[s, p], src_v, sem_b)
                st2 = pltpu.async_copy(dst_r.at[s, p], dst_v, sem_c)
                st1.wait()
                st2.wait()

        def pre_body(e, cnt):
            j = e // 8
            k = e - j * 8
            vd = dst_v[j, pl.ds(k * 16, 16)]
            vs = src_v[j, pl.ds(k * 16, 16)]

            @pl.when(c == 0)
            def _():
                plsc.addupdate_scatter(deg_v, [vd], ones16)

            local = vd - base
            valid = (local >= 0) & (local < HALF)
            plsc.store_compressed(dst_c.at[pl.ds(cnt, 16)], local, mask=valid)
            plsc.store_compressed(src_c.at[pl.ds(cnt, 16)], vs, mask=valid)
            return cnt + jnp.sum(jnp.where(valid, 1, 0))

        with jax.named_scope("compact"):
            cnt = lax.fori_loop(0, PHASE_CHUNKS * 8, pre_body, 0)

        for k in range(CHUNK // 16):
            dst_c[pl.ds(cnt + k * 16, 16)] = trash16
            src_c[pl.ds(cnt + k * 16, 16)] = zero16

        n_ch = (cnt + CHUNK - 1) // CHUNK

        def body(j, carry):
            pltpu.async_copy(h_hbm.at[src_c.at[pl.ds(j * CHUNK, CHUNK)]],
                             buf, sem).wait()
            pltpu.sync_copy(buf, acc.at[dst_c.at[pl.ds(j * CHUNK, CHUNK)]],
                            add=True)
            return carry

        with jax.named_scope("mainloop"):
            lax.fori_loop(0, n_ch, body, 0)

    with jax.named_scope("degout"):
        @pl.when(c == 0)
        def _():
            pltpu.sync_copy(deg_v, out_deg.at[s])

    plsc.subcore_barrier()
    with jax.named_scope("writeout"):
        pltpu.sync_copy(acc.at[pl.ds(row0, ROWS_PER_TILE)],
                        out_feat.at[c, pl.ds(row0, ROWS_PER_TILE)])


def _tc1_body(h_ref, wt1_ref, b_ref, o_ref):
    o_ref[...] = (
        jnp.dot(h_ref[...], wt1_ref[...], preferred_element_type=jnp.float32)
        + b_ref[...]
    )


def _tc2_body(t_ref, p_ref, deg_ref, wt2_ref, o_ref):
    summed = p_ref[0]
    deg = jnp.sum(deg_ref[...], axis=1)
    hn = summed / jnp.maximum(deg, 1.0)[:, None]
    o_ref[...] = t_ref[...] + jnp.dot(hn, wt2_ref[...],
                                      preferred_element_type=jnp.float32)


BLK = 1000


def kernel(h, edge_index, W, b):
    n, d = h.shape
    src = edge_index[0].astype(jnp.int32)
    dst = edge_index[1].astype(jnp.int32)
    pad = E_PAD - N_EDGES
    src = jnp.concatenate([src, jnp.zeros((pad,), jnp.int32)])
    dst = jnp.concatenate([dst, jnp.full((pad,), N_NODES, jnp.int32)])
    src = src.reshape(N_SUBCORES, N_PHASES, PHASE_CHUNKS, CHUNK)
    dst = dst.reshape(N_SUBCORES, N_PHASES, PHASE_CHUNKS, CHUNK)
    zeros2d = jnp.zeros((HALF_PAD, d), jnp.float32)
    zeros1d = jnp.zeros((DEG_PAD,), jnp.float32)

    partials, deg_parts = _sc_aggregate(h, src, dst, zeros2d, zeros1d)

    wt = W.T
    wt1 = wt[:d]
    wt2 = wt[d:]
    b2 = b.reshape(1, -1)

    tmp = pl.pallas_call(
        _tc1_body,
        grid=(n // BLK,),
        in_specs=[
            pl.BlockSpec((BLK, d), lambda i: (i, 0)),
            pl.BlockSpec((d, d), lambda i: (0, 0)),
            pl.BlockSpec((1, d), lambda i: (0, 0)),
        ],
        out_specs=pl.BlockSpec((BLK, d), lambda i: (i, 0)),
        out_shape=jax.ShapeDtypeStruct((n, d), jnp.float32),
    )(h, wt1, b2)

    nblk_half = HALF // BLK
    out = pl.pallas_call(
        _tc2_body,
        grid=(n // BLK,),
        in_specs=[
            pl.BlockSpec((BLK, d), lambda i: (i, 0)),
            pl.BlockSpec((1, BLK, d), lambda i: (i // nblk_half, i % nblk_half, 0)),
            pl.BlockSpec((BLK, N_SUBCORES), lambda i: (i, 0)),
            pl.BlockSpec((d, d), lambda i: (0, 0)),
        ],
        out_specs=pl.BlockSpec((BLK, d), lambda i: (i, 0)),
        out_shape=jax.ShapeDtypeStruct((n, d), jnp.float32),
    )(tmp, partials, deg_parts[:, :n].T, wt2)
    return out
